# baseline (speedup 1.0000x reference)
import functools
import math

import jax
import jax.numpy as jnp
from jax import lax
from jax.experimental import pallas as pl
from jax.experimental.pallas import tpu as pltpu
from jax.experimental.pallas import tpu_sc as plsc

_SCALES = (0.25, 0.125, 0.0625, 0.03125)
_DIMS = ((32, 64, 64), (16, 32, 32), (8, 16, 16), (4, 8, 8))
_BASES = (0, 131072, 147456, 149504)
_NROWS = 149760
_N = 512
_NBINS = 64
_NS = _N * _NBINS
_C = 64
_NW = 32
_SPW = _NS // _NW
_CH = 32
_NCH = _SPW // _CH


def _geom_body(braw_ref, idx_ref, w_ref):
    f32 = jnp.float32
    i32 = jnp.int32
    r = [braw_ref[i:i + 1, :] for i in range(7)]
    cx = r[0] * 200.0 + 28.0
    cy = r[1] * 200.0 + 28.0
    cz = r[2] * 80.0 + 20.0
    bw = 4.0 + r[3] * 192.0
    bl = 4.0 + r[4] * 192.0
    bh = 4.0 + r[5] * 20.0
    th = r[6] * (2.0 * math.pi)
    th = (th * (180.0 / math.pi)) * (math.pi / 180.0)
    ct = jnp.cos(th)
    st = jnp.sin(th)

    rate = jnp.sqrt(jnp.maximum(bw, bl)) / 56.0
    lvl = jnp.zeros((1, _N), dtype=i32)
    best = jnp.abs(_SCALES[0] - rate)
    for i in range(1, 4):
        d = jnp.abs(_SCALES[i] - rate)
        upd = d < best
        lvl = jnp.where(upd, i, lvl)
        best = jnp.where(upd, d, best)

    def sel(vals, dtype):
        out = jnp.full((1, _N), vals[3], dtype=dtype)
        for i in (2, 1, 0):
            out = jnp.where(lvl == i, dtype(vals[i]), out)
        return out

    ss = sel(_SCALES, f32)
    df = sel([d[0] for d in _DIMS], f32)
    hf = sel([d[1] for d in _DIMS], f32)
    wf = sel([d[2] for d in _DIMS], f32)
    hi = sel([d[1] for d in _DIMS], i32)
    wi = sel([d[2] for d in _DIMS], i32)
    base = sel(_BASES, i32)

    cyp = cy * ss
    cxp = cx * ss
    czp = cz * ss
    sy = bl * ss
    sx = bw * ss
    sz = bh * ss

    bin_i = lax.broadcasted_iota(i32, (_NBINS, _N), 0)
    gz = ((bin_i >> 4).astype(f32) + 0.5) * 0.25 - 0.5
    gy = (((bin_i >> 2) & 3).astype(f32) + 0.5) * 0.25 - 0.5
    gx = ((bin_i & 3).astype(f32) + 0.5) * 0.25 - 0.5

    ly = gy * sy
    lx = gx * sx
    lz = gz * sz
    yy = cyp + ly * ct - lx * st
    xx = cxp + ly * st + lx * ct
    zz = czp + lz

    z0 = jnp.floor(zz)
    y0 = jnp.floor(yy)
    x0 = jnp.floor(xx)
    wz = zz - z0
    wy = yy - y0
    wx = xx - x0

    pf = jnp.clip(x0, 0.0, wf - 1.0)
    rf = jnp.clip(x0 + 1.0, 0.0, wf - 1.0)
    qf = jnp.minimum(pf + 1.0, wf - 1.0)
    w_a = 1.0 - wx
    w_b = wx
    wl_x = w_a + jnp.where(rf == pf, w_b, 0.0)
    wu_x = jnp.where((rf == qf) & (qf != pf), w_b, 0.0)
    pi = pf.astype(i32)

    for dz in (0, 1):
        for dy in (0, 1):
            zi = jnp.clip(z0 + dz, 0.0, df - 1.0).astype(i32)
            yi = jnp.clip(y0 + dy, 0.0, hf - 1.0).astype(i32)
            lin = (zi * hi + yi) * wi + pi + base
            wzy = (wz if dz else 1.0 - wz) * (wy if dy else 1.0 - wy)
            pair = dz * 2 + dy
            idx_ref[pair, :, :] = lin
            w_ref[pair * 2, :, :] = wzy * wl_x
            w_ref[pair * 2 + 1, :, :] = wzy * wu_x


_geom = pl.pallas_call(
    _geom_body,
    out_shape=[
        jax.ShapeDtypeStruct((4, _NBINS, _N), jnp.int32),
        jax.ShapeDtypeStruct((8, _NBINS, _N), jnp.float32),
    ],
)


@functools.cache
def _make_sc_gather():
    mesh = plsc.VectorSubcoreMesh(core_axis_name="c", subcore_axis_name="s")

    @functools.partial(
        pl.kernel,
        mesh=mesh,
        out_type=jax.ShapeDtypeStruct((_NS * _C,), jnp.float32),
        scratch_types=[
            pltpu.VMEM((_SPW * 4,), jnp.int32),
            pltpu.VMEM((_SPW * 8,), jnp.float32),
            pltpu.VMEM((_CH * 4, 2 * _C), jnp.float32),
            pltpu.VMEM((_CH * 4, 2 * _C), jnp.float32),
            pltpu.VMEM((_SPW * _C,), jnp.float32),
            pltpu.SemaphoreType.DMA,
            pltpu.SemaphoreType.DMA,
        ],
    )
    def sc_gather(idx_hbm, w_hbm, table_hbm, out_hbm,
                  idx_v, w_v, rows0, rows1, out_v, sema, semb):
        wid = lax.axis_index("s") * 2 + lax.axis_index("c")
        sbase = wid * _SPW
        pltpu.sync_copy(idx_hbm.at[pl.ds(sbase * 4, _SPW * 4)], idx_v)
        pltpu.sync_copy(w_hbm.at[pl.ds(sbase * 8, _SPW * 8)], w_v)

        def start(c, buf, sem):
            return pltpu.async_copy(
                table_hbm.at[idx_v.at[pl.ds(c * (_CH * 4), _CH * 4)]],
                buf, sem)

        def drain(buf, sem):
            pltpu.make_async_copy(
                table_hbm.at[pl.ds(0, _CH * 4)], buf, sem).wait()

        def compute(c, buf):
            def pair(sp, carry):
                wvec = w_v[pl.ds(c * (_CH * 8) + sp * 16, 16)]
                for half in (0, 1):
                    s2 = sp * 2 + half
                    ws = [wvec[half * 8 + k] for k in range(8)]
                    for j in range(_C // 16):
                        acc = ws[0] * buf[s2 * 4, pl.ds(j * 16, 16)]
                        acc = acc + ws[1] * buf[s2 * 4, pl.ds(_C + j * 16, 16)]
                        for p in range(1, 4):
                            acc = acc + ws[2 * p] * buf[
                                s2 * 4 + p, pl.ds(j * 16, 16)]
                            acc = acc + ws[2 * p + 1] * buf[
                                s2 * 4 + p, pl.ds(_C + j * 16, 16)]
                        out_v[pl.ds((c * _CH + s2) * _C + j * 16, 16)] = acc
                return carry

            lax.fori_loop(0, _CH // 2, pair, 0)

        start(0, rows0, sema)

        def body(g, carry):
            c0 = g * 2
            h1 = start(c0 + 1, rows1, semb)
            drain(rows0, sema)
            compute(c0, rows0)
            start(jnp.minimum(c0 + 2, _NCH - 1), rows0, sema)
            h1.wait()
            compute(c0 + 1, rows1)
            return carry

        lax.fori_loop(0, _NCH // 2, body, 0)
        drain(rows0, sema)
        pltpu.sync_copy(out_v, out_hbm.at[pl.ds(sbase * _C, _SPW * _C)])

    return sc_gather


_TBLK = 8192
_TROWS = 19 * _TBLK


def _tbuild_body(f0_ref, f1_ref, f2_ref, f3_ref, out_ref):
    p = pl.program_id(0)
    eye = (lax.broadcasted_iota(jnp.int32, (_C, _C), 0) ==
           lax.broadcasted_iota(jnp.int32, (_C, _C), 1)).astype(jnp.float32)

    def emit(a_cs, w, rows, row_off):
        t = lax.dot_general(a_cs, eye, (((0,), (0,)), ((), ())),
                            precision=lax.Precision.HIGHEST,
                            preferred_element_type=jnp.float32)
        rolled = jnp.concatenate([t[1:], t[-1:]], axis=0)
        x = lax.broadcasted_iota(jnp.int32, (rows, 1), 0) & (w - 1)
        nxt = jnp.where(x == w - 1, t, rolled)
        out_ref[pl.ds(row_off, rows), 0:_C] = t
        out_ref[pl.ds(row_off, rows), _C:2 * _C] = nxt

    @pl.when(p < 16)
    def _():
        emit(f0_ref[0].reshape(_C, _TBLK), 64, _TBLK, 0)

    @pl.when((p == 16) | (p == 17))
    def _():
        emit(f1_ref[0].reshape(_C, _TBLK), 32, _TBLK, 0)

    @pl.when(p == 18)
    def _():
        emit(f2_ref[0].reshape(_C, 2048), 16, 2048, 0)
        emit(f3_ref[0].reshape(_C, 256), 8, 256, 2048)


_tbuild = pl.pallas_call(
    _tbuild_body,
    grid=(19,),
    in_specs=[
        pl.BlockSpec((1, _C, 2, 64, 64), lambda p: (0, 0, jnp.minimum(p, 15), 0, 0)),
        pl.BlockSpec((1, _C, 8, 32, 32),
                     lambda p: (0, 0, jnp.clip(p - 16, 0, 1), 0, 0)),
        pl.BlockSpec((1, _C, 8, 16, 16), lambda p: (0, 0, 0, 0, 0)),
        pl.BlockSpec((1, _C, 4, 8, 8), lambda p: (0, 0, 0, 0, 0)),
    ],
    out_specs=pl.BlockSpec((_TBLK, 2 * _C), lambda p: (p, 0)),
    out_shape=jax.ShapeDtypeStruct((_TROWS, 2 * _C), jnp.float32),
)


def _build_table(feat0, feat1, feat2, feat3):
    return _tbuild(feat0, feat1, feat2, feat3)


def kernel(feat0, feat1, feat2, feat3, boxes_raw):
    table = _build_table(feat0, feat1, feat2, feat3)
    idx4, w8 = _geom(boxes_raw.T)
    idx_flat = idx4.reshape(4, _NS).T.reshape(-1)
    w_flat = w8.reshape(8, _NS).T.reshape(-1)
    rows = _make_sc_gather()(idx_flat, w_flat, table)
    return rows.reshape(4, 4, 4, _N, _C).transpose(3, 4, 0, 1, 2)


# speedup vs baseline: 1.0836x; 1.0836x over previous
import functools
import math

import jax
import jax.numpy as jnp
from jax import lax
from jax.experimental import pallas as pl
from jax.experimental.pallas import tpu as pltpu
from jax.experimental.pallas import tpu_sc as plsc

_SCALES = (0.25, 0.125, 0.0625, 0.03125)
_DIMS = ((32, 64, 64), (16, 32, 32), (8, 16, 16), (4, 8, 8))
_BASES = (0, 131072, 147456, 149504)
_NROWS = 149760
_N = 512
_NBINS = 64
_NS = _N * _NBINS
_C = 64
_NW = 32
_SPW = _NS // _NW
_CH = 32
_NCH = _SPW // _CH


def _geom_body(braw_ref, idx_ref, w_ref):
    f32 = jnp.float32
    i32 = jnp.int32
    r = [braw_ref[i:i + 1, :] for i in range(7)]
    cx = r[0] * 200.0 + 28.0
    cy = r[1] * 200.0 + 28.0
    cz = r[2] * 80.0 + 20.0
    bw = 4.0 + r[3] * 192.0
    bl = 4.0 + r[4] * 192.0
    bh = 4.0 + r[5] * 20.0
    th = r[6] * (2.0 * math.pi)
    th = (th * (180.0 / math.pi)) * (math.pi / 180.0)
    ct = jnp.cos(th)
    st = jnp.sin(th)

    rate = jnp.sqrt(jnp.maximum(bw, bl)) / 56.0
    lvl = jnp.zeros((1, _N), dtype=i32)
    best = jnp.abs(_SCALES[0] - rate)
    for i in range(1, 4):
        d = jnp.abs(_SCALES[i] - rate)
        upd = d < best
        lvl = jnp.where(upd, i, lvl)
        best = jnp.where(upd, d, best)

    def sel(vals, dtype):
        out = jnp.full((1, _N), vals[3], dtype=dtype)
        for i in (2, 1, 0):
            out = jnp.where(lvl == i, dtype(vals[i]), out)
        return out

    ss = sel(_SCALES, f32)
    df = sel([d[0] for d in _DIMS], f32)
    hf = sel([d[1] for d in _DIMS], f32)
    wf = sel([d[2] for d in _DIMS], f32)
    hi = sel([d[1] for d in _DIMS], i32)
    wi = sel([d[2] for d in _DIMS], i32)
    base = sel(_BASES, i32)

    cyp = cy * ss
    cxp = cx * ss
    czp = cz * ss
    sy = bl * ss
    sx = bw * ss
    sz = bh * ss

    bin_i = lax.broadcasted_iota(i32, (_NBINS, _N), 0)
    gz = ((bin_i >> 4).astype(f32) + 0.5) * 0.25 - 0.5
    gy = (((bin_i >> 2) & 3).astype(f32) + 0.5) * 0.25 - 0.5
    gx = ((bin_i & 3).astype(f32) + 0.5) * 0.25 - 0.5

    ly = gy * sy
    lx = gx * sx
    lz = gz * sz
    yy = cyp + ly * ct - lx * st
    xx = cxp + ly * st + lx * ct
    zz = czp + lz

    z0 = jnp.floor(zz)
    y0 = jnp.floor(yy)
    x0 = jnp.floor(xx)
    wz = zz - z0
    wy = yy - y0
    wx = xx - x0

    pf = jnp.clip(x0, 0.0, wf - 1.0)
    rf = jnp.clip(x0 + 1.0, 0.0, wf - 1.0)
    qf = jnp.minimum(pf + 1.0, wf - 1.0)
    w_a = 1.0 - wx
    w_b = wx
    wl_x = w_a + jnp.where(rf == pf, w_b, 0.0)
    wu_x = jnp.where((rf == qf) & (qf != pf), w_b, 0.0)
    pi = pf.astype(i32)

    for dz in (0, 1):
        for dy in (0, 1):
            zi = jnp.clip(z0 + dz, 0.0, df - 1.0).astype(i32)
            yi = jnp.clip(y0 + dy, 0.0, hf - 1.0).astype(i32)
            lin = (zi * hi + yi) * wi + pi + base
            wzy = (wz if dz else 1.0 - wz) * (wy if dy else 1.0 - wy)
            pair = dz * 2 + dy
            idx_ref[pair, :, :] = lin
            w_ref[pair * 2, :, :] = wzy * wl_x
            w_ref[pair * 2 + 1, :, :] = wzy * wu_x


_geom = pl.pallas_call(
    _geom_body,
    out_shape=[
        jax.ShapeDtypeStruct((4, _NBINS, _N), jnp.int32),
        jax.ShapeDtypeStruct((8, _NBINS, _N), jnp.float32),
    ],
)


@functools.cache
def _make_sc_gather():
    mesh = plsc.VectorSubcoreMesh(core_axis_name="c", subcore_axis_name="s")

    @functools.partial(
        pl.kernel,
        mesh=mesh,
        out_type=jax.ShapeDtypeStruct((_NS * _C,), jnp.float32),
        scratch_types=[
            pltpu.VMEM((_SPW * 4,), jnp.int32),
            pltpu.VMEM((_SPW * 8,), jnp.float32),
            pltpu.VMEM((_CH * 4, 2 * _C), jnp.float32),
            pltpu.VMEM((_CH * 4, 2 * _C), jnp.float32),
            pltpu.VMEM((_SPW * _C,), jnp.float32),
            pltpu.SemaphoreType.DMA,
            pltpu.SemaphoreType.DMA,
        ],
    )
    def sc_gather(idx_hbm, w_hbm, table_hbm, out_hbm,
                  idx_v, w_v, rows0, rows1, out_v, sema, semb):
        wid = lax.axis_index("s") * 2 + lax.axis_index("c")
        sbase = wid * _SPW
        pltpu.sync_copy(idx_hbm.at[pl.ds(sbase * 4, _SPW * 4)], idx_v)
        pltpu.sync_copy(w_hbm.at[pl.ds(sbase * 8, _SPW * 8)], w_v)

        def start(c, buf, sem):
            return pltpu.async_copy(
                table_hbm.at[idx_v.at[pl.ds(c * (_CH * 4), _CH * 4)]],
                buf, sem)

        def drain(buf, sem):
            pltpu.make_async_copy(
                table_hbm.at[pl.ds(0, _CH * 4)], buf, sem).wait()

        def compute(c, buf):
            def pair(sp, carry):
                wvec = w_v[pl.ds(c * (_CH * 8) + sp * 16, 16)]
                for half in (0, 1):
                    s2 = sp * 2 + half
                    ws = [wvec[half * 8 + k] for k in range(8)]
                    for j in range(_C // 16):
                        acc = ws[0] * buf[s2 * 4, pl.ds(j * 16, 16)]
                        acc = acc + ws[1] * buf[s2 * 4, pl.ds(_C + j * 16, 16)]
                        for p in range(1, 4):
                            acc = acc + ws[2 * p] * buf[
                                s2 * 4 + p, pl.ds(j * 16, 16)]
                            acc = acc + ws[2 * p + 1] * buf[
                                s2 * 4 + p, pl.ds(_C + j * 16, 16)]
                        out_v[pl.ds((c * _CH + s2) * _C + j * 16, 16)] = acc
                return carry

            lax.fori_loop(0, _CH // 2, pair, 0)

        start(0, rows0, sema)

        def body(g, carry):
            c0 = g * 2
            h1 = start(c0 + 1, rows1, semb)
            drain(rows0, sema)
            compute(c0, rows0)
            start(jnp.minimum(c0 + 2, _NCH - 1), rows0, sema)
            h1.wait()
            compute(c0 + 1, rows1)
            return carry

        lax.fori_loop(0, _NCH // 2, body, 0)
        drain(rows0, sema)
        pltpu.sync_copy(out_v, out_hbm.at[pl.ds(sbase * _C, _SPW * _C)])

    return sc_gather


_TBLK = 8192
_TROWS = 19 * _TBLK


def _tbuild_body(f0_ref, f1_ref, f2_ref, f3_ref, out_ref):
    p = pl.program_id(0)
    eye = (lax.broadcasted_iota(jnp.int32, (_C, _C), 0) ==
           lax.broadcasted_iota(jnp.int32, (_C, _C), 1)).astype(jnp.float32)

    def emit(a_cs, w, rows, row_off):
        t = jnp.transpose(a_cs)
        rolled = jnp.concatenate([t[1:], t[-1:]], axis=0)
        x = lax.broadcasted_iota(jnp.int32, (rows, 1), 0) & (w - 1)
        nxt = jnp.where(x == w - 1, t, rolled)
        out_ref[pl.ds(row_off, rows), 0:_C] = t
        out_ref[pl.ds(row_off, rows), _C:2 * _C] = nxt

    @pl.when(p < 16)
    def _():
        emit(f0_ref[0].reshape(_C, _TBLK), 64, _TBLK, 0)

    @pl.when((p == 16) | (p == 17))
    def _():
        emit(f1_ref[0].reshape(_C, _TBLK), 32, _TBLK, 0)

    @pl.when(p == 18)
    def _():
        emit(f2_ref[0].reshape(_C, 2048), 16, 2048, 0)
        emit(f3_ref[0].reshape(_C, 256), 8, 256, 2048)


_tbuild = pl.pallas_call(
    _tbuild_body,
    grid=(19,),
    in_specs=[
        pl.BlockSpec((1, _C, 2, 64, 64), lambda p: (0, 0, jnp.minimum(p, 15), 0, 0)),
        pl.BlockSpec((1, _C, 8, 32, 32),
                     lambda p: (0, 0, jnp.clip(p - 16, 0, 1), 0, 0)),
        pl.BlockSpec((1, _C, 8, 16, 16), lambda p: (0, 0, 0, 0, 0)),
        pl.BlockSpec((1, _C, 4, 8, 8), lambda p: (0, 0, 0, 0, 0)),
    ],
    out_specs=pl.BlockSpec((_TBLK, 2 * _C), lambda p: (p, 0)),
    out_shape=jax.ShapeDtypeStruct((_TROWS, 2 * _C), jnp.float32),
)


def _build_table(feat0, feat1, feat2, feat3):
    return _tbuild(feat0, feat1, feat2, feat3)


def kernel(feat0, feat1, feat2, feat3, boxes_raw):
    table = _build_table(feat0, feat1, feat2, feat3)
    idx4, w8 = _geom(boxes_raw.T)
    idx_flat = idx4.reshape(4, _NS).T.reshape(-1)
    w_flat = w8.reshape(8, _NS).T.reshape(-1)
    rows = _make_sc_gather()(idx_flat, w_flat, table)
    return rows.reshape(4, 4, 4, _N, _C).transpose(3, 4, 0, 1, 2)


# speedup vs baseline: 1.1594x; 1.0700x over previous
import functools
import math

import jax
import jax.numpy as jnp
from jax import lax
from jax.experimental import pallas as pl
from jax.experimental.pallas import tpu as pltpu
from jax.experimental.pallas import tpu_sc as plsc

_SCALES = (0.25, 0.125, 0.0625, 0.03125)
_DIMS = ((32, 64, 64), (16, 32, 32), (8, 16, 16), (4, 8, 8))
_BASES = (0, 131072, 147456, 149504)
_NROWS = 149760
_N = 512
_NBINS = 64
_NS = _N * _NBINS
_C = 64
_NW = 32
_SPW = _NS // _NW
_CH = 32
_NCH = _SPW // _CH


def _geom_body(braw_ref, idx_ref, w_ref):
    f32 = jnp.float32
    i32 = jnp.int32
    r = [braw_ref[i:i + 1, :] for i in range(7)]
    cx = r[0] * 200.0 + 28.0
    cy = r[1] * 200.0 + 28.0
    cz = r[2] * 80.0 + 20.0
    bw = 4.0 + r[3] * 192.0
    bl = 4.0 + r[4] * 192.0
    bh = 4.0 + r[5] * 20.0
    th = r[6] * (2.0 * math.pi)
    th = (th * (180.0 / math.pi)) * (math.pi / 180.0)
    ct = jnp.cos(th)
    st = jnp.sin(th)

    rate = jnp.sqrt(jnp.maximum(bw, bl)) / 56.0
    lvl = jnp.zeros((1, _N), dtype=i32)
    best = jnp.abs(_SCALES[0] - rate)
    for i in range(1, 4):
        d = jnp.abs(_SCALES[i] - rate)
        upd = d < best
        lvl = jnp.where(upd, i, lvl)
        best = jnp.where(upd, d, best)

    def sel(vals, dtype):
        out = jnp.full((1, _N), vals[3], dtype=dtype)
        for i in (2, 1, 0):
            out = jnp.where(lvl == i, dtype(vals[i]), out)
        return out

    ss = sel(_SCALES, f32)
    df = sel([d[0] for d in _DIMS], f32)
    hf = sel([d[1] for d in _DIMS], f32)
    wf = sel([d[2] for d in _DIMS], f32)
    hi = sel([d[1] for d in _DIMS], i32)
    wi = sel([d[2] for d in _DIMS], i32)
    base = sel(_BASES, i32)

    cyp = cy * ss
    cxp = cx * ss
    czp = cz * ss
    sy = bl * ss
    sx = bw * ss
    sz = bh * ss

    bin_i = lax.broadcasted_iota(i32, (_NBINS, _N), 0)
    gz = ((bin_i >> 4).astype(f32) + 0.5) * 0.25 - 0.5
    gy = (((bin_i >> 2) & 3).astype(f32) + 0.5) * 0.25 - 0.5
    gx = ((bin_i & 3).astype(f32) + 0.5) * 0.25 - 0.5

    ly = gy * sy
    lx = gx * sx
    lz = gz * sz
    yy = cyp + ly * ct - lx * st
    xx = cxp + ly * st + lx * ct
    zz = czp + lz

    z0 = jnp.floor(zz)
    y0 = jnp.floor(yy)
    x0 = jnp.floor(xx)
    wz = zz - z0
    wy = yy - y0
    wx = xx - x0

    pf = jnp.clip(x0, 0.0, wf - 1.0)
    rf = jnp.clip(x0 + 1.0, 0.0, wf - 1.0)
    qf = jnp.minimum(pf + 1.0, wf - 1.0)
    w_a = 1.0 - wx
    w_b = wx
    wl_x = w_a + jnp.where(rf == pf, w_b, 0.0)
    wu_x = jnp.where((rf == qf) & (qf != pf), w_b, 0.0)
    pi = pf.astype(i32)

    for dz in (0, 1):
        for dy in (0, 1):
            zi = jnp.clip(z0 + dz, 0.0, df - 1.0).astype(i32)
            yi = jnp.clip(y0 + dy, 0.0, hf - 1.0).astype(i32)
            lin = (zi * hi + yi) * wi + pi + base
            wzy = (wz if dz else 1.0 - wz) * (wy if dy else 1.0 - wy)
            pair = dz * 2 + dy
            idx_ref[pair, :, :] = lin
            w_ref[pair * 2, :, :] = wzy * wl_x
            w_ref[pair * 2 + 1, :, :] = wzy * wu_x


_geom = pl.pallas_call(
    _geom_body,
    out_shape=[
        jax.ShapeDtypeStruct((4, _NBINS, _N), jnp.int32),
        jax.ShapeDtypeStruct((8, _NBINS, _N), jnp.float32),
    ],
)


@functools.cache
def _make_sc_gather():
    mesh = plsc.VectorSubcoreMesh(core_axis_name="c", subcore_axis_name="s")

    @functools.partial(
        pl.kernel,
        mesh=mesh,
        out_type=jax.ShapeDtypeStruct((_NS * _C,), jnp.float32),
        scratch_types=[
            pltpu.VMEM((_SPW * 4,), jnp.int32),
            pltpu.VMEM((_SPW * 8,), jnp.float32),
            pltpu.VMEM((_CH * 4, 2 * _C), jnp.float32),
            pltpu.VMEM((_CH * 4, 2 * _C), jnp.float32),
            pltpu.VMEM((_SPW * _C,), jnp.float32),
            pltpu.SemaphoreType.DMA,
            pltpu.SemaphoreType.DMA,
        ],
    )
    def sc_gather(idx_hbm, w_hbm, table_hbm, out_hbm,
                  idx_v, w_v, rows0, rows1, out_v, sema, semb):
        wid = lax.axis_index("s") * 2 + lax.axis_index("c")
        sbase = wid * _SPW
        pltpu.sync_copy(idx_hbm.at[pl.ds(sbase * 4, _SPW * 4)], idx_v)
        pltpu.sync_copy(w_hbm.at[pl.ds(sbase * 8, _SPW * 8)], w_v)

        def start(c, buf, sem):
            return pltpu.async_copy(
                table_hbm.at[idx_v.at[pl.ds(c * (_CH * 4), _CH * 4)]],
                buf, sem)

        def drain(buf, sem):
            pltpu.make_async_copy(
                table_hbm.at[pl.ds(0, _CH * 4)], buf, sem).wait()

        def compute(c, buf):
            @plsc.parallel_loop(0, _CH // 2, unroll=2)
            def pair(sp):
                wvec = w_v[pl.ds(c * (_CH * 8) + sp * 16, 16)]
                for half in (0, 1):
                    s2 = sp * 2 + half
                    ws = [wvec[half * 8 + k] for k in range(8)]
                    for j in range(_C // 16):
                        acc = ws[0] * buf[s2 * 4, pl.ds(j * 16, 16)]
                        acc = acc + ws[1] * buf[s2 * 4, pl.ds(_C + j * 16, 16)]
                        for p in range(1, 4):
                            acc = acc + ws[2 * p] * buf[
                                s2 * 4 + p, pl.ds(j * 16, 16)]
                            acc = acc + ws[2 * p + 1] * buf[
                                s2 * 4 + p, pl.ds(_C + j * 16, 16)]
                        out_v[pl.ds((c * _CH + s2) * _C + j * 16, 16)] = acc

        start(0, rows0, sema)

        def body(g, carry):
            c0 = g * 2
            h1 = start(c0 + 1, rows1, semb)
            drain(rows0, sema)
            compute(c0, rows0)
            start(jnp.minimum(c0 + 2, _NCH - 1), rows0, sema)
            h1.wait()
            compute(c0 + 1, rows1)
            return carry

        lax.fori_loop(0, _NCH // 2, body, 0)
        drain(rows0, sema)
        pltpu.sync_copy(out_v, out_hbm.at[pl.ds(sbase * _C, _SPW * _C)])

    return sc_gather


_TBLK = 8192
_TROWS = 19 * _TBLK


def _tbuild_body(f0_ref, f1_ref, f2_ref, f3_ref, out_ref):
    p = pl.program_id(0)
    eye = (lax.broadcasted_iota(jnp.int32, (_C, _C), 0) ==
           lax.broadcasted_iota(jnp.int32, (_C, _C), 1)).astype(jnp.float32)

    def emit(a_cs, w, rows, row_off):
        t = jnp.transpose(a_cs)
        rolled = jnp.concatenate([t[1:], t[-1:]], axis=0)
        x = lax.broadcasted_iota(jnp.int32, (rows, 1), 0) & (w - 1)
        nxt = jnp.where(x == w - 1, t, rolled)
        out_ref[pl.ds(row_off, rows), 0:_C] = t
        out_ref[pl.ds(row_off, rows), _C:2 * _C] = nxt

    @pl.when(p < 16)
    def _():
        emit(f0_ref[0].reshape(_C, _TBLK), 64, _TBLK, 0)

    @pl.when((p == 16) | (p == 17))
    def _():
        emit(f1_ref[0].reshape(_C, _TBLK), 32, _TBLK, 0)

    @pl.when(p == 18)
    def _():
        emit(f2_ref[0].reshape(_C, 2048), 16, 2048, 0)
        emit(f3_ref[0].reshape(_C, 256), 8, 256, 2048)


_tbuild = pl.pallas_call(
    _tbuild_body,
    grid=(19,),
    in_specs=[
        pl.BlockSpec((1, _C, 2, 64, 64), lambda p: (0, 0, jnp.minimum(p, 15), 0, 0)),
        pl.BlockSpec((1, _C, 8, 32, 32),
                     lambda p: (0, 0, jnp.clip(p - 16, 0, 1), 0, 0)),
        pl.BlockSpec((1, _C, 8, 16, 16), lambda p: (0, 0, 0, 0, 0)),
        pl.BlockSpec((1, _C, 4, 8, 8), lambda p: (0, 0, 0, 0, 0)),
    ],
    out_specs=pl.BlockSpec((_TBLK, 2 * _C), lambda p: (p, 0)),
    out_shape=jax.ShapeDtypeStruct((_TROWS, 2 * _C), jnp.float32),
)


def _build_table(feat0, feat1, feat2, feat3):
    return _tbuild(feat0, feat1, feat2, feat3)


def kernel(feat0, feat1, feat2, feat3, boxes_raw):
    table = _build_table(feat0, feat1, feat2, feat3)
    idx4, w8 = _geom(boxes_raw.T)
    idx_flat = idx4.reshape(4, _NS).T.reshape(-1)
    w_flat = w8.reshape(8, _NS).T.reshape(-1)
    rows = _make_sc_gather()(idx_flat, w_flat, table)
    return rows.reshape(4, 4, 4, _N, _C).transpose(3, 4, 0, 1, 2)
